# speedup vs baseline: 1.2053x; 1.2053x over previous
import jax
import jax.numpy as jnp
from jax.experimental import pallas as pl
from jax.experimental.pallas import tpu as pltpu

N_NODES = 10000
N_EDGES = 320000
D_IN = 128
HID = 128
EDGE_DIM = 16
OUT = 1
NUM_LAYERS = 3
NUM_GRAPHS = 64


def _head_kernel(g_ref, w1_ref, b1_ref, w2_ref, b2_ref, o_ref):
    g = g_ref[...]
    z = jnp.maximum(jnp.dot(g, w1_ref[...], preferred_element_type=jnp.float32)
                    + b1_ref[...], 0.0)
    o_ref[...] = jnp.dot(z, w2_ref[...], preferred_element_type=jnp.float32) + b2_ref[...]


def kernel(x, edge_index, edge_attr, batch, W, att_src, att_dst, W_edge,
           att_edge, bias, W1, b1, W2, b2):
    src = edge_index[0].astype(jnp.int32)
    dst = edge_index[1].astype(jnp.int32)
    batch = batch.astype(jnp.int32)

    V = jnp.einsum("leh,lh->el", W_edge, att_edge)
    alpha_e_all = edge_attr @ V

    h = x
    for l in range(NUM_LAYERS):
        h128 = h @ W[l]
        a_s = h128 @ att_src[l]
        a_d = h128 @ att_dst[l]
        alpha = a_s[src] + a_d[dst] + alpha_e_all[:, l]
        alpha = jax.nn.leaky_relu(alpha, negative_slope=0.2)
        shift_n = jax.nn.leaky_relu(a_d + (jnp.max(a_s) + jnp.max(alpha_e_all[:, l])),
                                    negative_slope=0.2)
        ex = jnp.exp(alpha - shift_n[dst])
        msg = jnp.concatenate([h128[src] * ex[:, None], ex[:, None]], axis=1)
        acc = jax.ops.segment_sum(msg, dst, num_segments=N_NODES)
        h = jnp.maximum(acc[:, :HID] / (acc[:, HID:] + 1e-16) + bias[l], 0.0)

    gmax = jax.ops.segment_max(h, batch, num_segments=NUM_GRAPHS)
    gmax = jnp.where(jnp.isfinite(gmax), gmax, 0.0)
    gsum = jax.ops.segment_sum(h, batch, num_segments=NUM_GRAPHS)
    cnt = jax.ops.segment_sum(jnp.ones((N_NODES,), jnp.float32), batch,
                              num_segments=NUM_GRAPHS)
    gmean = gsum / jnp.maximum(cnt, 1.0)[:, None]
    g = jnp.concatenate([gmax, gmean], axis=1)

    out = pl.pallas_call(
        _head_kernel,
        out_shape=jax.ShapeDtypeStruct((NUM_GRAPHS, OUT), jnp.float32),
    )(g, W1, b1, W2, b2)
    return out


# speedup vs baseline: 12.2296x; 10.1462x over previous
import dataclasses
import functools

import jax
import jax.numpy as jnp
from jax import lax
from jax.experimental import pallas as pl
from jax.experimental.pallas import tpu as pltpu
from jax.experimental.pallas import tpu_sc as plsc

N_NODES = 10000
N_EDGES = 320000
HID = 128
EDGE_DIM = 16
NUM_LAYERS = 3
NUM_GRAPHS = 64

NP = 10240
SINK = NP - 1
NC, NS, LANES = 2, 16, 16
NW = NC * NS
CH = 64
EPW = 10112
E_PAD = EPW * NW
DEN_ROWS = NP // HID
ACC_R = NP + HID
ROWS_PER_SUB = ACC_R // NS


def _edge_proj_kernel(ea_ref, we_ref, ae_att_ref, out_ref, mx_ref):
    cols = []
    for l in range(NUM_LAYERS):
        cols.append(jnp.dot(we_ref[l], ae_att_ref[l],
                            preferred_element_type=jnp.float32))
    v8 = jnp.stack(cols + [jnp.zeros((EDGE_DIM,), jnp.float32)] * 5, axis=1)
    blk = lax.dot_general(v8, ea_ref[...], (((0,), (1,)), ((), ())),
                          preferred_element_type=jnp.float32)
    out_ref[...] = blk
    bm = jnp.max(blk, axis=1, keepdims=True)
    bmx = jnp.broadcast_to(bm, (8, 16))

    @pl.when(pl.program_id(0) == 0)
    def _():
        mx_ref[...] = bmx

    @pl.when(pl.program_id(0) != 0)
    def _():
        mx_ref[...] = jnp.maximum(mx_ref[...], bmx)


def _edge_proj(edge_attr_pad, W_edge, att_edge):
    nblk = 16
    blk = E_PAD // nblk
    return pl.pallas_call(
        _edge_proj_kernel,
        grid=(nblk,),
        in_specs=[
            pl.BlockSpec((blk, EDGE_DIM), lambda i: (i, 0)),
            pl.BlockSpec((NUM_LAYERS, EDGE_DIM, HID), lambda i: (0, 0, 0)),
            pl.BlockSpec((NUM_LAYERS, HID), lambda i: (0, 0)),
        ],
        out_specs=[
            pl.BlockSpec((8, blk), lambda i: (0, i)),
            pl.BlockSpec((8, 16), lambda i: (0, 0)),
        ],
        out_shape=[
            jax.ShapeDtypeStruct((8, E_PAD), jnp.float32),
            jax.ShapeDtypeStruct((8, 16), jnp.float32),
        ],
    )(edge_attr_pad, W_edge, att_edge)


def _prep_kernel(h_ref, w_ref, asrc_ref, adst_ref, aemx_ref,
                 h128_ref, a_s_ref, a_d_ref, m16_ref):
    h128 = jnp.dot(h_ref[...], w_ref[...], preferred_element_type=jnp.float32)
    h128_ref[...] = h128
    a_s = jnp.sum(h128 * asrc_ref[...][None, :], axis=1)
    a_d = jnp.sum(h128 * adst_ref[...][None, :], axis=1)
    a_s_ref[...] = a_s
    a_d_ref[...] = a_d
    blk_max = jnp.broadcast_to(jnp.max(a_s), (16,))

    @pl.when(pl.program_id(0) == 0)
    def _():
        m16_ref[...] = blk_max + aemx_ref[...]

    @pl.when(pl.program_id(0) != 0)
    def _():
        m16_ref[...] = jnp.maximum(m16_ref[...], blk_max + aemx_ref[...])


def _prep(h, w, asrc, adst, aemx16):
    nblk = 5
    blk = NP // nblk
    return pl.pallas_call(
        _prep_kernel,
        grid=(nblk,),
        in_specs=[
            pl.BlockSpec((blk, HID), lambda i: (i, 0)),
            pl.BlockSpec((HID, HID), lambda i: (0, 0)),
            pl.BlockSpec((HID,), lambda i: (0,)),
            pl.BlockSpec((HID,), lambda i: (0,)),
            pl.BlockSpec((16,), lambda i: (0,)),
        ],
        out_specs=[
            pl.BlockSpec((blk, HID), lambda i: (i, 0)),
            pl.BlockSpec((blk,), lambda i: (i,)),
            pl.BlockSpec((blk,), lambda i: (i,)),
            pl.BlockSpec((16,), lambda i: (0,)),
        ],
        out_shape=[
            jax.ShapeDtypeStruct((NP, HID), jnp.float32),
            jax.ShapeDtypeStruct((NP,), jnp.float32),
            jax.ShapeDtypeStruct((NP,), jnp.float32),
            jax.ShapeDtypeStruct((16,), jnp.float32),
        ],
    )(h, w, asrc, adst, aemx16)


def _expand_den(den2d, nrows):
    na = den2d.shape[0]
    i0 = lax.broadcasted_iota(jnp.int32, (nrows, na), 0)
    a = lax.broadcasted_iota(jnp.int32, (nrows, na), 1)
    oh1 = (lax.shift_right_logical(i0, 7) == a).astype(jnp.float32)
    t = lax.dot_general(oh1, den2d, (((1,), (0,)), ((), ())),
                        preferred_element_type=jnp.float32)
    i0b = lax.broadcasted_iota(jnp.int32, (nrows, HID), 0)
    b = lax.broadcasted_iota(jnp.int32, (nrows, HID), 1)
    oh2 = (jnp.bitwise_and(i0b, 127) == b).astype(jnp.float32)
    return jnp.sum(t * oh2, axis=1, keepdims=True)


def _finalize_kernel(msg_ref, den_ref, b_ref, h_ref):
    s = msg_ref[0] + msg_ref[1]
    den = _expand_den(den_ref[0] + den_ref[1], s.shape[0])
    h = s / (den + 1e-16) + b_ref[...][None, :]
    h_ref[...] = jnp.maximum(h, 0.0)


def _finalize(acc, b):
    nblk = 5
    blk = NP // nblk
    dblk = blk // HID
    return pl.pallas_call(
        _finalize_kernel,
        grid=(nblk,),
        in_specs=[
            pl.BlockSpec((2, blk, HID), lambda i: (0, i, 0)),
            pl.BlockSpec((2, dblk, HID), lambda i: (0, NP // dblk + i, 0)),
            pl.BlockSpec((HID,), lambda i: (0,)),
        ],
        out_specs=pl.BlockSpec((blk, HID), lambda i: (i, 0)),
        out_shape=jax.ShapeDtypeStruct((NP, HID), jnp.float32),
    )(acc, acc, b)


def _sc_edge(h128, a_s, a_d, m16, src, dst, ae_l):
    mesh = plsc.VectorSubcoreMesh(core_axis_name="c", subcore_axis_name="s")
    cp = pltpu.CompilerParams()
    if "needs_layout_passes" in pltpu.CompilerParams.__dataclass_fields__:
        cp = dataclasses.replace(cp, needs_layout_passes=False)

    @functools.partial(
        pl.kernel,
        out_type=jax.ShapeDtypeStruct((2, ACC_R, HID), jnp.float32),
        mesh=mesh,
        compiler_params=cp,
        scratch_types=[
            pltpu.VMEM((NP,), jnp.float32),
            pltpu.VMEM((NP,), jnp.float32),
            pltpu.VMEM((16,), jnp.float32),
            pltpu.VMEM((CH,), jnp.int32),
            pltpu.VMEM((CH,), jnp.int32),
            pltpu.VMEM((CH,), jnp.int32),
            pltpu.VMEM((CH,), jnp.float32),
            pltpu.VMEM((CH,), jnp.float32),
            pltpu.VMEM((CH, HID), jnp.float32),
            pltpu.VMEM((CH, HID), jnp.float32),
            pltpu.VMEM_SHARED((ACC_R, HID), jnp.float32),
            pltpu.SemaphoreType.DMA,
        ],
    )
    def k(h128_hbm, a_s_hbm, a_d_hbm, m16_hbm, src_hbm, dst_hbm, ae_hbm,
          acc_out, asv, adv, m16v, srcv, dstv, exdstv, aev, exv, rows,
          stg_ex, acc_sh, sem):
        cid = lax.axis_index("c")
        sid = lax.axis_index("s")
        wid = sid * NC + cid
        base = wid * EPW

        pltpu.sync_copy(a_s_hbm, asv)
        pltpu.sync_copy(a_d_hbm, adv)
        pltpu.sync_copy(m16_hbm, m16v)

        @pl.loop(0, CH)
        def _(kk):
            for j in range(HID // LANES):
                stg_ex[kk, pl.ds(j * LANES, LANES)] = jnp.zeros(
                    (LANES,), jnp.float32)

        zbase = sid * ROWS_PER_SUB
        for r in range(ROWS_PER_SUB // CH):
            pltpu.sync_copy(stg_ex, acc_sh.at[pl.ds(zbase + r * CH, CH)])
        rem = ROWS_PER_SUB % CH
        if rem:
            pltpu.sync_copy(stg_ex.at[pl.ds(0, rem)],
                            acc_sh.at[pl.ds(zbase + ROWS_PER_SUB - rem, rem)])
        plsc.subcore_barrier()

        mreg = m16v[...]

        @pl.loop(0, EPW // CH)
        def _(q):
            off = base + q * CH
            pltpu.sync_copy(src_hbm.at[pl.ds(off, CH)], srcv)
            pltpu.sync_copy(dst_hbm.at[pl.ds(off, CH)], dstv)
            pltpu.sync_copy(ae_hbm.at[pl.ds(off, CH)], aev)
            pltpu.async_copy(h128_hbm.at[srcv], rows, sem).wait()

            @pl.loop(0, CH // LANES)
            def _(j):
                s16 = srcv[pl.ds(j * LANES, LANES)]
                d16 = dstv[pl.ds(j * LANES, LANES)]
                as16 = plsc.load_gather(asv, [s16])
                ad16 = plsc.load_gather(adv, [d16])
                ae16 = aev[pl.ds(j * LANES, LANES)]
                t = as16 + ad16 + ae16
                alpha = jnp.maximum(t, 0.2 * t)
                sh = ad16 + mreg
                shift = jnp.maximum(sh, 0.2 * sh)
                ex16 = jnp.exp(alpha - shift)
                exv[pl.ds(j * LANES, LANES)] = ex16
                exdstv[pl.ds(j * LANES, LANES)] = (
                    jax.lax.shift_right_logical(d16, 7) + NP)
                idx0 = jax.lax.iota(jnp.int32, LANES) + j * LANES
                ec16 = jax.lax.bitwise_and(d16, 127)
                plsc.store_scatter(stg_ex, [idx0, ec16], ex16)

            @pl.loop(0, CH)
            def _(kk):
                exk16 = plsc.load_gather(
                    exv, [jnp.full((LANES,), kk, jnp.int32)])
                for j2 in range(HID // LANES):
                    rows[kk, pl.ds(j2 * LANES, LANES)] = (
                        rows[kk, pl.ds(j2 * LANES, LANES)] * exk16)

            pltpu.sync_copy(rows, acc_sh.at[dstv], add=True)
            pltpu.sync_copy(stg_ex, acc_sh.at[exdstv], add=True)

            @pl.loop(0, CH // LANES)
            def _(j):
                d16 = dstv[pl.ds(j * LANES, LANES)]
                idx0 = jax.lax.iota(jnp.int32, LANES) + j * LANES
                ec16 = jax.lax.bitwise_and(d16, 127)
                plsc.store_scatter(stg_ex, [idx0, ec16],
                                   jnp.zeros((LANES,), jnp.float32))

        plsc.subcore_barrier()
        pltpu.sync_copy(
            acc_sh.at[pl.ds(sid * ROWS_PER_SUB, ROWS_PER_SUB)],
            acc_out.at[cid, pl.ds(sid * ROWS_PER_SUB, ROWS_PER_SUB)])

    return k(h128, a_s, a_d, m16, src, dst, ae_l)


def _head_kernel(acc_ref, b_ref, batch_ref, w1_ref, b1_ref, w2_ref, b2_ref,
                 o_ref, gmax_s):
    s = acc_ref[0, :NP, :] + acc_ref[1, :NP, :]
    den = _expand_den(acc_ref[0, NP:NP + DEN_ROWS, :]
                      + acc_ref[1, NP:NP + DEN_ROWS, :], NP)
    h = jnp.maximum(s / (den + 1e-16) + b_ref[...][None, :], 0.0)
    batch_col = batch_ref[...].reshape(NP, 1)

    gi = lax.broadcasted_iota(jnp.int32, (NUM_GRAPHS, NP), 0)
    oh = (gi == batch_ref[...][None, :]).astype(jnp.float32)
    gsum = lax.dot_general(oh, h, (((1,), (0,)), ((), ())),
                           preferred_element_type=jnp.float32)
    cnt = jnp.sum(oh, axis=1).reshape(NUM_GRAPHS, 1)

    def body(g, _):
        mask = (batch_col == g)
        t = jnp.where(mask, h, 0.0)
        gm = jnp.max(t, axis=0)
        gmax_s[pl.ds(g, 1), :] = gm[None, :]
        return 0

    lax.fori_loop(0, NUM_GRAPHS, body, 0)
    gmax = gmax_s[...]
    gmean = gsum / jnp.maximum(cnt, 1.0)
    g256 = jnp.concatenate([gmax, gmean], axis=1)
    z = jnp.maximum(
        jnp.dot(g256, w1_ref[...], preferred_element_type=jnp.float32)
        + b1_ref[...][None, :], 0.0)
    o_ref[...] = (jnp.dot(z, w2_ref[...], preferred_element_type=jnp.float32)
                  + b2_ref[...][None, :])


def _head(acc, b, batch_pad, W1, b1, W2, b2):
    return pl.pallas_call(
        _head_kernel,
        out_shape=jax.ShapeDtypeStruct((NUM_GRAPHS, 1), jnp.float32),
        scratch_shapes=[pltpu.VMEM((NUM_GRAPHS, HID), jnp.float32)],
    )(acc, b, batch_pad, W1, b1, W2, b2)


def kernel(x, edge_index, edge_attr, batch, W, att_src, att_dst, W_edge,
           att_edge, bias, W1, b1, W2, b2):
    src = edge_index[0].astype(jnp.int32)
    dst = edge_index[1].astype(jnp.int32)
    pad_e = E_PAD - N_EDGES
    src_p = jnp.concatenate([src, jnp.zeros((pad_e,), jnp.int32)])
    dst_p = jnp.concatenate([dst, jnp.full((pad_e,), SINK, jnp.int32)])
    ea_p = jnp.concatenate(
        [edge_attr, jnp.zeros((pad_e, EDGE_DIM), jnp.float32)], axis=0)
    x_p = jnp.concatenate(
        [x, jnp.zeros((NP - N_NODES, x.shape[1]), jnp.float32)], axis=0)
    batch_p = jnp.concatenate(
        [batch.astype(jnp.int32),
         jnp.full((NP - N_NODES,), NUM_GRAPHS, jnp.int32)])

    ae_all, ae_mx = _edge_proj(ea_p, W_edge, att_edge)

    h = x_p
    acc = None
    for l in range(NUM_LAYERS):
        if l > 0:
            h = _finalize(acc, bias[l - 1])
        h128, a_s, a_d, m16 = _prep(h, W[l], att_src[l], att_dst[l],
                                    ae_mx[l])
        acc = _sc_edge(h128, a_s, a_d, m16, src_p, dst_p, ae_all[l])

    return _head(acc, bias[NUM_LAYERS - 1], batch_p, W1, b1, W2, b2)


# speedup vs baseline: 17.9304x; 1.4662x over previous
import dataclasses
import functools

import jax
import jax.numpy as jnp
from jax import lax
from jax.experimental import pallas as pl
from jax.experimental.pallas import tpu as pltpu
from jax.experimental.pallas import tpu_sc as plsc

N_NODES = 10000
N_EDGES = 320000
HID = 128
EDGE_DIM = 16
NUM_LAYERS = 3
NUM_GRAPHS = 64

NP = 10240
SINK = NP - 1
NC, NS, LANES = 2, 16, 16
NW = NC * NS
CH = 64
EPW = 10112
E_PAD = EPW * NW
DEN_ROWS = NP // HID
ACC_R = NP + HID
ROWS_PER_SUB = ACC_R // NS


def _edge_proj_kernel(ea_ref, we_ref, ae_att_ref, out_ref, mx_ref):
    cols = []
    for l in range(NUM_LAYERS):
        cols.append(jnp.dot(we_ref[l], ae_att_ref[l],
                            preferred_element_type=jnp.float32))
    v8 = jnp.stack(cols + [jnp.zeros((EDGE_DIM,), jnp.float32)] * 5, axis=1)
    blk = lax.dot_general(v8, ea_ref[...], (((0,), (1,)), ((), ())),
                          preferred_element_type=jnp.float32)
    out_ref[...] = blk
    bm = jnp.max(blk, axis=1, keepdims=True)
    bmx = jnp.broadcast_to(bm, (8, 16))

    @pl.when(pl.program_id(0) == 0)
    def _():
        mx_ref[...] = bmx

    @pl.when(pl.program_id(0) != 0)
    def _():
        mx_ref[...] = jnp.maximum(mx_ref[...], bmx)


def _edge_proj(edge_attr_pad, W_edge, att_edge):
    nblk = 16
    blk = E_PAD // nblk
    return pl.pallas_call(
        _edge_proj_kernel,
        grid=(nblk,),
        in_specs=[
            pl.BlockSpec((blk, EDGE_DIM), lambda i: (i, 0)),
            pl.BlockSpec((NUM_LAYERS, EDGE_DIM, HID), lambda i: (0, 0, 0)),
            pl.BlockSpec((NUM_LAYERS, HID), lambda i: (0, 0)),
        ],
        out_specs=[
            pl.BlockSpec((8, blk), lambda i: (0, i)),
            pl.BlockSpec((8, 16), lambda i: (0, 0)),
        ],
        out_shape=[
            jax.ShapeDtypeStruct((8, E_PAD), jnp.float32),
            jax.ShapeDtypeStruct((8, 16), jnp.float32),
        ],
    )(edge_attr_pad, W_edge, att_edge)


def _prep_kernel(h_ref, w_ref, asrc_ref, adst_ref, aemx_ref,
                 h128_ref, a_s_ref, a_d_ref, m16_ref):
    h128 = jnp.dot(h_ref[...], w_ref[...], preferred_element_type=jnp.float32)
    h128_ref[...] = h128
    a_s = jnp.sum(h128 * asrc_ref[...][None, :], axis=1)
    a_d = jnp.sum(h128 * adst_ref[...][None, :], axis=1)
    a_s_ref[...] = a_s
    a_d_ref[...] = a_d
    blk_max = jnp.broadcast_to(jnp.max(a_s), (16,))

    @pl.when(pl.program_id(0) == 0)
    def _():
        m16_ref[...] = blk_max + aemx_ref[...]

    @pl.when(pl.program_id(0) != 0)
    def _():
        m16_ref[...] = jnp.maximum(m16_ref[...], blk_max + aemx_ref[...])


def _prep(h, w, asrc, adst, aemx16):
    nblk = 5
    blk = NP // nblk
    return pl.pallas_call(
        _prep_kernel,
        grid=(nblk,),
        in_specs=[
            pl.BlockSpec((blk, HID), lambda i: (i, 0)),
            pl.BlockSpec((HID, HID), lambda i: (0, 0)),
            pl.BlockSpec((HID,), lambda i: (0,)),
            pl.BlockSpec((HID,), lambda i: (0,)),
            pl.BlockSpec((16,), lambda i: (0,)),
        ],
        out_specs=[
            pl.BlockSpec((blk, HID), lambda i: (i, 0)),
            pl.BlockSpec((blk,), lambda i: (i,)),
            pl.BlockSpec((blk,), lambda i: (i,)),
            pl.BlockSpec((16,), lambda i: (0,)),
        ],
        out_shape=[
            jax.ShapeDtypeStruct((NP, HID), jnp.float32),
            jax.ShapeDtypeStruct((NP,), jnp.float32),
            jax.ShapeDtypeStruct((NP,), jnp.float32),
            jax.ShapeDtypeStruct((16,), jnp.float32),
        ],
    )(h, w, asrc, adst, aemx16)


def _expand_den(den2d, nrows):
    na = den2d.shape[0]
    i0 = lax.broadcasted_iota(jnp.int32, (nrows, na), 0)
    a = lax.broadcasted_iota(jnp.int32, (nrows, na), 1)
    oh1 = (lax.shift_right_logical(i0, 7) == a).astype(jnp.float32)
    t = lax.dot_general(oh1, den2d, (((1,), (0,)), ((), ())),
                        preferred_element_type=jnp.float32)
    i0b = lax.broadcasted_iota(jnp.int32, (nrows, HID), 0)
    b = lax.broadcasted_iota(jnp.int32, (nrows, HID), 1)
    oh2 = (jnp.bitwise_and(i0b, 127) == b).astype(jnp.float32)
    return jnp.sum(t * oh2, axis=1, keepdims=True)


def _finalize_kernel(msg_ref, den_ref, b_ref, h_ref):
    s = msg_ref[0] + msg_ref[1]
    den = _expand_den(den_ref[0] + den_ref[1], s.shape[0])
    h = s / (den + 1e-16) + b_ref[...][None, :]
    h_ref[...] = jnp.maximum(h, 0.0)


def _finalize(acc, b):
    nblk = 5
    blk = NP // nblk
    dblk = blk // HID
    return pl.pallas_call(
        _finalize_kernel,
        grid=(nblk,),
        in_specs=[
            pl.BlockSpec((2, blk, HID), lambda i: (0, i, 0)),
            pl.BlockSpec((2, dblk, HID), lambda i: (0, NP // dblk + i, 0)),
            pl.BlockSpec((HID,), lambda i: (0,)),
        ],
        out_specs=pl.BlockSpec((blk, HID), lambda i: (i, 0)),
        out_shape=jax.ShapeDtypeStruct((NP, HID), jnp.float32),
    )(acc, acc, b)


def _sc_edge(h128, a_s, a_d, m16, src, dst, ae_l):
    mesh = plsc.VectorSubcoreMesh(core_axis_name="c", subcore_axis_name="s")
    cp = pltpu.CompilerParams()
    if "needs_layout_passes" in pltpu.CompilerParams.__dataclass_fields__:
        cp = dataclasses.replace(cp, needs_layout_passes=False)

    @functools.partial(
        pl.kernel,
        out_type=jax.ShapeDtypeStruct((2, ACC_R, HID), jnp.float32),
        mesh=mesh,
        compiler_params=cp,
        scratch_types=[
            pltpu.VMEM((NP,), jnp.float32),
            pltpu.VMEM((NP,), jnp.float32),
            pltpu.VMEM((16,), jnp.float32),
            pltpu.VMEM((CH,), jnp.int32),
            pltpu.VMEM((CH,), jnp.int32),
            pltpu.VMEM((CH,), jnp.int32),
            pltpu.VMEM((CH,), jnp.int32),
            pltpu.VMEM((CH,), jnp.float32),
            pltpu.VMEM((CH,), jnp.float32),
            pltpu.VMEM((CH,), jnp.int32),
            pltpu.VMEM((CH,), jnp.int32),
            pltpu.VMEM((CH,), jnp.float32),
            pltpu.VMEM((CH, HID), jnp.float32),
            pltpu.VMEM((CH, HID), jnp.float32),
            pltpu.VMEM((CH, HID), jnp.float32),
            pltpu.VMEM_SHARED((ACC_R, HID), jnp.float32),
            pltpu.SemaphoreType.DMA,
            pltpu.SemaphoreType.DMA,
            pltpu.SemaphoreType.DMA,
            pltpu.SemaphoreType.DMA,
        ],
    )
    def k(h128_hbm, a_s_hbm, a_d_hbm, m16_hbm, src_hbm, dst_hbm, ae_hbm,
          acc_out, asv, adv, m16v, srcv0, srcv1, dstv0, dstv1, aev0, aev1,
          dstw, exdstv, exv, rows0, rows1, stg_ex, acc_sh,
          rsem0, rsem1, gsem0, gsem1):
        cid = lax.axis_index("c")
        sid = lax.axis_index("s")
        wid = sid * NC + cid
        base = wid * EPW

        bufs = ((srcv0, dstv0, aev0, rows0, rsem0, gsem0),
                (srcv1, dstv1, aev1, rows1, rsem1, gsem1))

        pltpu.sync_copy(a_s_hbm, asv)
        pltpu.sync_copy(a_d_hbm, adv)
        pltpu.sync_copy(m16_hbm, m16v)

        @pl.loop(0, CH)
        def _(kk):
            for j in range(HID // LANES):
                stg_ex[kk, pl.ds(j * LANES, LANES)] = jnp.zeros(
                    (LANES,), jnp.float32)

        zbase = sid * ROWS_PER_SUB
        for r in range(ROWS_PER_SUB // CH):
            pltpu.sync_copy(stg_ex, acc_sh.at[pl.ds(zbase + r * CH, CH)])
        rem = ROWS_PER_SUB % CH
        if rem:
            pltpu.sync_copy(stg_ex.at[pl.ds(0, rem)],
                            acc_sh.at[pl.ds(zbase + ROWS_PER_SUB - rem, rem)])
        plsc.subcore_barrier()

        mreg = m16v[...]

        def issue_rec(q, p):
            sv, dv, av, _, rs, _ = bufs[p]
            off = base + q * CH
            pltpu.async_copy(src_hbm.at[pl.ds(off, CH)], sv, rs)
            pltpu.async_copy(dst_hbm.at[pl.ds(off, CH)], dv, rs)
            pltpu.async_copy(ae_hbm.at[pl.ds(off, CH)], av, rs)

        def wait_rec(p):
            sv, dv, av, _, rs, _ = bufs[p]
            pltpu.make_async_copy(src_hbm.at[pl.ds(0, CH)], sv, rs).wait()
            pltpu.make_async_copy(dst_hbm.at[pl.ds(0, CH)], dv, rs).wait()
            pltpu.make_async_copy(ae_hbm.at[pl.ds(0, CH)], av, rs).wait()

        def issue_gather(p):
            sv, _, _, rw, _, gs = bufs[p]
            pltpu.async_copy(h128_hbm.at[sv], rw, gs)

        def wait_gather(p):
            sv, _, _, rw, _, gs = bufs[p]
            pltpu.make_async_copy(h128_hbm.at[sv], rw, gs).wait()

        def ex_compute(p):
            sv, dv, av = bufs[p][0], bufs[p][1], bufs[p][2]

            @pl.loop(0, CH // LANES)
            def _(j):
                s16 = sv[pl.ds(j * LANES, LANES)]
                d16 = dv[pl.ds(j * LANES, LANES)]
                as16 = plsc.load_gather(asv, [s16])
                ad16 = plsc.load_gather(adv, [d16])
                ae16 = av[pl.ds(j * LANES, LANES)]
                t = as16 + ad16 + ae16
                alpha = jnp.maximum(t, 0.2 * t)
                sh = ad16 + mreg
                shift = jnp.maximum(sh, 0.2 * sh)
                ex16 = jnp.exp(alpha - shift)
                exv[pl.ds(j * LANES, LANES)] = ex16
                dstw[pl.ds(j * LANES, LANES)] = d16
                exdstv[pl.ds(j * LANES, LANES)] = (
                    jax.lax.shift_right_logical(d16, 7) + NP)
                idx0 = jax.lax.iota(jnp.int32, LANES) + j * LANES
                ec16 = jax.lax.bitwise_and(d16, 127)
                plsc.store_scatter(stg_ex, [idx0, ec16], ex16)

        def scale_scatter(p):
            rw = bufs[p][3]

            @pl.loop(0, CH)
            def _(kk):
                exk16 = plsc.load_gather(
                    exv, [jnp.full((LANES,), kk, jnp.int32)])
                for j2 in range(HID // LANES):
                    rw[kk, pl.ds(j2 * LANES, LANES)] = (
                        rw[kk, pl.ds(j2 * LANES, LANES)] * exk16)

            pltpu.sync_copy(rw, acc_sh.at[dstw], add=True)
            pltpu.sync_copy(stg_ex, acc_sh.at[exdstv], add=True)

            @pl.loop(0, CH // LANES)
            def _(j):
                idx0 = jax.lax.iota(jnp.int32, LANES) + j * LANES
                ec16 = jax.lax.bitwise_and(dstw[pl.ds(j * LANES, LANES)], 127)
                plsc.store_scatter(stg_ex, [idx0, ec16],
                                   jnp.zeros((LANES,), jnp.float32))

        def pair(a, with_rec):
            wait_rec(0)
            ga = pltpu.async_copy(h128_hbm.at[bufs[0][0]], bufs[0][3], gsem0)
            ex_compute(0)
            wait_rec(1)
            gb = pltpu.async_copy(h128_hbm.at[bufs[1][0]], bufs[1][3], gsem1)
            ga.wait()
            if with_rec:
                issue_rec(a + 2, 0)
            scale_scatter(0)
            ex_compute(1)
            gb.wait()
            if with_rec:
                issue_rec(a + 3, 1)
            scale_scatter(1)

        nchunks = EPW // CH
        issue_rec(0, 0)
        issue_rec(1, 1)

        @pl.loop(0, nchunks // 2 - 1)
        def _(i):
            pair(2 * i, True)

        pair(nchunks - 2, False)

        plsc.subcore_barrier()
        pltpu.sync_copy(
            acc_sh.at[pl.ds(sid * ROWS_PER_SUB, ROWS_PER_SUB)],
            acc_out.at[cid, pl.ds(sid * ROWS_PER_SUB, ROWS_PER_SUB)])

    return k(h128, a_s, a_d, m16, src, dst, ae_l)


def _head_kernel(acc_ref, b_ref, batch_ref, w1_ref, b1_ref, w2_ref, b2_ref,
                 o_ref, gmax_s):
    s = acc_ref[0, :NP, :] + acc_ref[1, :NP, :]
    den = _expand_den(acc_ref[0, NP:NP + DEN_ROWS, :]
                      + acc_ref[1, NP:NP + DEN_ROWS, :], NP)
    h = jnp.maximum(s / (den + 1e-16) + b_ref[...][None, :], 0.0)
    batch_col = batch_ref[...].reshape(NP, 1)

    gi = lax.broadcasted_iota(jnp.int32, (NUM_GRAPHS, NP), 0)
    oh = (gi == batch_ref[...][None, :]).astype(jnp.float32)
    gsum = lax.dot_general(oh, h, (((1,), (0,)), ((), ())),
                           preferred_element_type=jnp.float32)
    cnt = jnp.sum(oh, axis=1).reshape(NUM_GRAPHS, 1)

    def body(g, _):
        mask = (batch_col == g)
        t = jnp.where(mask, h, 0.0)
        gm = jnp.max(t, axis=0)
        gmax_s[pl.ds(g, 1), :] = gm[None, :]
        return 0

    lax.fori_loop(0, NUM_GRAPHS, body, 0)
    gmax = gmax_s[...]
    gmean = gsum / jnp.maximum(cnt, 1.0)
    g256 = jnp.concatenate([gmax, gmean], axis=1)
    z = jnp.maximum(
        jnp.dot(g256, w1_ref[...], preferred_element_type=jnp.float32)
        + b1_ref[...][None, :], 0.0)
    o_ref[...] = (jnp.dot(z, w2_ref[...], preferred_element_type=jnp.float32)
                  + b2_ref[...][None, :])


def _head(acc, b, batch_pad, W1, b1, W2, b2):
    return pl.pallas_call(
        _head_kernel,
        out_shape=jax.ShapeDtypeStruct((NUM_GRAPHS, 1), jnp.float32),
        scratch_shapes=[pltpu.VMEM((NUM_GRAPHS, HID), jnp.float32)],
    )(acc, b, batch_pad, W1, b1, W2, b2)


def kernel(x, edge_index, edge_attr, batch, W, att_src, att_dst, W_edge,
           att_edge, bias, W1, b1, W2, b2):
    src = edge_index[0].astype(jnp.int32)
    dst = edge_index[1].astype(jnp.int32)
    pad_e = E_PAD - N_EDGES
    src_p = jnp.concatenate([src, jnp.zeros((pad_e,), jnp.int32)])
    dst_p = jnp.concatenate([dst, jnp.full((pad_e,), SINK, jnp.int32)])
    ea_p = jnp.concatenate(
        [edge_attr, jnp.zeros((pad_e, EDGE_DIM), jnp.float32)], axis=0)
    x_p = jnp.concatenate(
        [x, jnp.zeros((NP - N_NODES, x.shape[1]), jnp.float32)], axis=0)
    batch_p = jnp.concatenate(
        [batch.astype(jnp.int32),
         jnp.full((NP - N_NODES,), NUM_GRAPHS, jnp.int32)])

    ae_all, ae_mx = _edge_proj(ea_p, W_edge, att_edge)

    h = x_p
    acc = None
    for l in range(NUM_LAYERS):
        if l > 0:
            h = _finalize(acc, bias[l - 1])
        h128, a_s, a_d, m16 = _prep(h, W[l], att_src[l], att_dst[l],
                                    ae_mx[l])
        acc = _sc_edge(h128, a_s, a_d, m16, src_p, dst_p, ae_all[l])

    return _head(acc, bias[NUM_LAYERS - 1], batch_p, W1, b1, W2, b2)
